# baseline (speedup 1.0000x reference)
import dataclasses
import functools

import jax
import jax.numpy as jnp
from jax import lax
from jax.experimental import pallas as pl
from jax.experimental.pallas import tpu as pltpu
from jax.experimental.pallas import tpu_sc as plsc

_NUM_CORES = 2
_NUM_SUBCORES = 16
_NUM_WORKERS = _NUM_CORES * _NUM_SUBCORES
_WINDOW = 128
_TAIL_PAD = 144
_BLK = 4096


def _exp_split_body(a_ref, d_ref, ea_ref, ed_ref, ta_ref, td_ref):
    ea_ref[...] = jnp.exp(a_ref[:, :128])
    ed_ref[...] = jnp.exp(d_ref[:, :128])
    pad = jnp.zeros((_TAIL_PAD - 129,), jnp.float32)
    ta_ref[...] = jnp.concatenate([jnp.exp(a_ref[:, 128]), pad])
    td_ref[...] = jnp.concatenate([jnp.exp(d_ref[:, 128]), pad])


def _exp_split(log_alpha_w, log_delta_w):
    v = log_alpha_w.shape[0]
    main = jax.ShapeDtypeStruct((v, 128), jnp.float32)
    tail = jax.ShapeDtypeStruct((_TAIL_PAD,), jnp.float32)
    return pl.pallas_call(
        _exp_split_body, out_shape=(main, main, tail, tail)
    )(log_alpha_w, log_delta_w)


def _make_gather(n, n_per_w):
    mesh = plsc.VectorSubcoreMesh(core_axis_name="c", subcore_axis_name="s")
    main = jax.ShapeDtypeStruct((n, 128), jnp.float32)
    tail = jax.ShapeDtypeStruct((n // 128, 128), jnp.float32)
    cp = pltpu.CompilerParams()
    if "needs_layout_passes" in pltpu.CompilerParams.__dataclass_fields__:
        cp = dataclasses.replace(cp, needs_layout_passes=False)
    nsteps = n_per_w // _WINDOW

    @functools.partial(
        pl.kernel,
        mesh=mesh,
        compiler_params=cp,
        out_type=(main, main, tail, tail),
        scratch_types=[
            pltpu.VMEM((2, _WINDOW), jnp.int32),
            pltpu.VMEM((2, _WINDOW, 128), jnp.float32),
            pltpu.VMEM((2, _WINDOW, 128), jnp.float32),
            pltpu.VMEM((2, _WINDOW), jnp.float32),
            pltpu.VMEM((2, _WINDOW), jnp.float32),
            pltpu.VMEM((_TAIL_PAD,), jnp.float32),
            pltpu.VMEM((_TAIL_PAD,), jnp.float32),
            pltpu.SemaphoreType.DMA((2,)),
            pltpu.SemaphoreType.DMA((2,)),
        ],
    )
    def gather_kernel(ea_hbm, ed_hbm, ta_hbm, td_hbm, idx_hbm,
                      ma_hbm, md_hbm, t2a_hbm, t2d_hbm,
                      idx_v, ba_v, bd_v, sa_v, sd_v, ta_v, td_v,
                      sem_g, sem_w):
        wid = lax.axis_index("s") * _NUM_CORES + lax.axis_index("c")
        base = wid * n_per_w
        pltpu.sync_copy(ta_hbm, ta_v)
        pltpu.sync_copy(td_hbm, td_v)

        def drain_writes(p, start):
            pltpu.make_async_copy(
                ba_v.at[p], ma_hbm.at[pl.ds(start, _WINDOW)],
                sem_w.at[p]).wait()
            pltpu.make_async_copy(
                bd_v.at[p], md_hbm.at[pl.ds(start, _WINDOW)],
                sem_w.at[p]).wait()
            pltpu.make_async_copy(
                sa_v.at[p], t2a_hbm.at[start // 128], sem_w.at[p]).wait()
            pltpu.make_async_copy(
                sd_v.at[p], t2d_hbm.at[start // 128], sem_w.at[p]).wait()

        @pl.loop(0, nsteps, step=2)
        def _(g0):
            handles = []
            for p in (0, 1):
                g = g0 + p
                start = base + g * _WINDOW

                @pl.when(g >= 2)
                def _():
                    drain_writes(p, start)

                pltpu.sync_copy(idx_hbm.at[pl.ds(start, _WINDOW)],
                                idx_v.at[p])
                ha = pltpu.async_copy(
                    ea_hbm.at[idx_v.at[p]], ba_v.at[p], sem_g.at[p])
                hd = pltpu.async_copy(
                    ed_hbm.at[idx_v.at[p]], bd_v.at[p], sem_g.at[p])
                handles.append((ha, hd))

            for p in (0, 1):
                g = g0 + p
                start = base + g * _WINDOW
                for k in range(_WINDOW // 16):
                    idxs = idx_v.at[p][pl.ds(16 * k, 16)]
                    sa_v.at[p][pl.ds(16 * k, 16)] = plsc.load_gather(
                        ta_v, [idxs])
                    sd_v.at[p][pl.ds(16 * k, 16)] = plsc.load_gather(
                        td_v, [idxs])
                ha, hd = handles[p]
                ha.wait()
                hd.wait()
                pltpu.async_copy(ba_v.at[p], ma_hbm.at[pl.ds(start, _WINDOW)],
                                 sem_w.at[p])
                pltpu.async_copy(bd_v.at[p], md_hbm.at[pl.ds(start, _WINDOW)],
                                 sem_w.at[p])
                pltpu.async_copy(sa_v.at[p], t2a_hbm.at[start // 128],
                                 sem_w.at[p])
                pltpu.async_copy(sd_v.at[p], t2d_hbm.at[start // 128],
                                 sem_w.at[p])

        for p in (0, 1):
            start = base + p * _WINDOW
            drain_writes(p, start)

    return gather_kernel


def _assemble_body(ma_ref, md_ref, t2a_ref, t2d_ref, oa_ref, od_ref):
    oa_ref[:, :128] = ma_ref[...]
    od_ref[:, :128] = md_ref[...]
    tta = jnp.transpose(t2a_ref[...])
    ttd = jnp.transpose(t2d_ref[...])
    for g in range(_BLK // 128):
        oa_ref[pl.ds(128 * g, 128), 128:129] = tta[:, g:g + 1]
        od_ref[pl.ds(128 * g, 128), 128:129] = ttd[:, g:g + 1]


def _assemble(n, ma, md, t2a, t2d):
    grid = (n // _BLK,)
    out = jax.ShapeDtypeStruct((n, 129), jnp.float32)
    main_spec = pl.BlockSpec((_BLK, 128), lambda i: (i, 0))
    tail_spec = pl.BlockSpec((_BLK // 128, 128), lambda i: (i, 0))
    out_spec = pl.BlockSpec((_BLK, 129), lambda i: (i, 0))
    return pl.pallas_call(
        _assemble_body,
        grid=grid,
        in_specs=[main_spec, main_spec, tail_spec, tail_spec],
        out_specs=(out_spec, out_spec),
        out_shape=(out, out),
    )(ma, md, t2a, t2d)


def kernel(events, log_alpha_w, log_delta_w):
    b, s = events.shape
    v, d = log_alpha_w.shape
    n = b * s

    ea, ed, ta, td = _exp_split(log_alpha_w, log_delta_w)
    idx = events.reshape(n).astype(jnp.int32)

    n_per_w = n // _NUM_WORKERS
    ma, md, t2a, t2d = _make_gather(n, n_per_w)(ea, ed, ta, td, idx)
    oa, od = _assemble(n, ma, md, t2a, t2d)
    return oa.reshape(b, s, d), od.reshape(b, s, d)


# speedup vs baseline: 1.6466x; 1.6466x over previous
import dataclasses
import functools

import jax
import jax.numpy as jnp
from jax import lax
from jax.experimental import pallas as pl
from jax.experimental.pallas import tpu as pltpu
from jax.experimental.pallas import tpu_sc as plsc

_NUM_CORES = 2
_NUM_SUBCORES = 16
_NUM_WORKERS = _NUM_CORES * _NUM_SUBCORES
_WINDOW = 80
_TAIL_PAD = 144


def _exp_split_body(a_ref, d_ref, ea_ref, ed_ref, ta_ref, td_ref):
    ea = jnp.exp(a_ref[:, :128])
    ed = jnp.exp(d_ref[:, :128])
    for r in range(_NUM_WORKERS):
        ea_ref[pl.ds(129 * r, 129), :] = ea
        ed_ref[pl.ds(129 * r, 129), :] = ed
    pad = jnp.zeros((_TAIL_PAD - 129,), jnp.float32)
    ta_ref[...] = jnp.concatenate([jnp.exp(a_ref[:, 128]), pad])
    td_ref[...] = jnp.concatenate([jnp.exp(d_ref[:, 128]), pad])


def _exp_split(log_alpha_w, log_delta_w):
    v = log_alpha_w.shape[0]
    main = jax.ShapeDtypeStruct((v * _NUM_WORKERS, 128), jnp.float32)
    tail = jax.ShapeDtypeStruct((_TAIL_PAD,), jnp.float32)
    return pl.pallas_call(
        _exp_split_body, out_shape=(main, main, tail, tail)
    )(log_alpha_w, log_delta_w)


def _make_gather(n, d, n_per_w):
    mesh = plsc.VectorSubcoreMesh(core_axis_name="c", subcore_axis_name="s")
    out = jax.ShapeDtypeStruct((n, d), jnp.float32)
    cp = pltpu.CompilerParams()
    if "needs_layout_passes" in pltpu.CompilerParams.__dataclass_fields__:
        cp = dataclasses.replace(cp, needs_layout_passes=False)
    nsteps = n_per_w // _WINDOW

    @functools.partial(
        pl.kernel,
        mesh=mesh,
        compiler_params=cp,
        out_type=(out, out),
        scratch_types=[
            pltpu.VMEM((2, _WINDOW), jnp.int32),
            pltpu.VMEM((2, _WINDOW, d), jnp.float32),
            pltpu.VMEM((2, _WINDOW, d), jnp.float32),
            pltpu.VMEM((_TAIL_PAD,), jnp.float32),
            pltpu.VMEM((_TAIL_PAD,), jnp.float32),
            pltpu.SemaphoreType.DMA((2,)),
            pltpu.SemaphoreType.DMA((2,)),
        ],
    )
    def gather_kernel(ea_hbm, ed_hbm, ta_hbm, td_hbm, idx_hbm, oa_hbm, od_hbm,
                      idx_v, ca_v, cd_v, ta_v, td_v, sem_g, sem_w):
        wid = lax.axis_index("s") * _NUM_CORES + lax.axis_index("c")
        base = wid * n_per_w
        rebase = jnp.broadcast_to(wid * 129, (16,)).astype(jnp.int32)
        pltpu.sync_copy(ta_hbm, ta_v)
        pltpu.sync_copy(td_hbm, td_v)
        last_col = jnp.full((16,), d - 1, jnp.int32)

        def fixup_tail(p):
            for k in range(_WINDOW // 16):
                rows = lax.iota(jnp.int32, 16) + (16 * k)
                idxs = idx_v.at[p][pl.ds(16 * k, 16)]
                va = plsc.load_gather(ta_v, [idxs])
                vd = plsc.load_gather(td_v, [idxs])
                plsc.store_scatter(ca_v.at[p], [rows, last_col], va)
                plsc.store_scatter(cd_v.at[p], [rows, last_col], vd)

        @pl.loop(0, nsteps, step=2)
        def _(g0):
            handles = []
            for p in (0, 1):
                g = g0 + p
                start = base + g * _WINDOW

                @pl.when(g >= 2)
                def _():
                    pltpu.make_async_copy(
                        ca_v.at[p], oa_hbm.at[pl.ds(start, _WINDOW)],
                        sem_w.at[p]).wait()
                    pltpu.make_async_copy(
                        cd_v.at[p], od_hbm.at[pl.ds(start, _WINDOW)],
                        sem_w.at[p]).wait()

                pltpu.sync_copy(idx_hbm.at[pl.ds(start, _WINDOW)],
                                idx_v.at[p])
                fixup_tail(p)
                for k in range(_WINDOW // 16):
                    sl = pl.ds(16 * k, 16)
                    idx_v.at[p][sl] = idx_v.at[p][sl] + rebase
                ha = pltpu.async_copy(
                    ea_hbm.at[idx_v.at[p]],
                    ca_v.at[p].at[:, pl.ds(0, d - 1)], sem_g.at[p])
                hd = pltpu.async_copy(
                    ed_hbm.at[idx_v.at[p]],
                    cd_v.at[p].at[:, pl.ds(0, d - 1)], sem_g.at[p])
                handles.append((ha, hd))

            for p in (0, 1):
                g = g0 + p
                start = base + g * _WINDOW
                ha, hd = handles[p]
                ha.wait()
                hd.wait()
                pltpu.async_copy(ca_v.at[p], oa_hbm.at[pl.ds(start, _WINDOW)],
                                 sem_w.at[p])
                pltpu.async_copy(cd_v.at[p], od_hbm.at[pl.ds(start, _WINDOW)],
                                 sem_w.at[p])

        for p in (0, 1):
            start = base + p * _WINDOW
            pltpu.make_async_copy(
                ca_v.at[p], oa_hbm.at[pl.ds(start, _WINDOW)],
                sem_w.at[p]).wait()
            pltpu.make_async_copy(
                cd_v.at[p], od_hbm.at[pl.ds(start, _WINDOW)],
                sem_w.at[p]).wait()

    return gather_kernel


def kernel(events, log_alpha_w, log_delta_w):
    b, s = events.shape
    v, d = log_alpha_w.shape
    n = b * s

    ea, ed, ta, td = _exp_split(log_alpha_w, log_delta_w)
    idx = events.reshape(n).astype(jnp.int32)

    n_per_w = n // _NUM_WORKERS
    oa, od = _make_gather(n, d, n_per_w)(ea, ed, ta, td, idx)
    return oa.reshape(b, s, d), od.reshape(b, s, d)
